# baseline (speedup 1.0000x reference)
import functools

import jax
import jax.numpy as jnp
from jax import lax
from jax.experimental import pallas as pl
from jax.experimental.pallas import tpu as pltpu
from jax.experimental.pallas import tpu_sc as plsc

B = 16384
H = 128
NC = 2
NS = 16
NW = NC * NS
ROWS_W = B // NW
CH = 128
NCH = ROWS_W // CH

BT = 2048


def _gather_body(tt_hbm, tid_hbm, st_hbm, sid_hbm, t_out, s_out,
                 tidx_v, sidx_v, rows_v, dsem):
  wid = lax.axis_index("s") * NC + lax.axis_index("c")
  base = wid * ROWS_W
  pltpu.sync_copy(sid_hbm.at[wid], sidx_v)
  pltpu.sync_copy(tid_hbm.at[wid], tidx_v)
  waits = []
  for c in range(NCH):
    waits.append(pltpu.async_copy(
        st_hbm.at[sidx_v.at[c]], rows_v.at[pl.ds(c * CH, CH)], dsem))
  for w in waits:
    w.wait()
  pltpu.sync_copy(rows_v, s_out.at[pl.ds(base, ROWS_W)])
  waits = []
  for c in range(NCH):
    waits.append(pltpu.async_copy(
        tt_hbm.at[tidx_v.at[c]], rows_v.at[pl.ds(c * CH, CH)], dsem))
  for w in waits:
    w.wait()
  pltpu.sync_copy(rows_v, t_out.at[pl.ds(base, ROWS_W)])


@functools.cache
def _gather_call():
  return pl.kernel(
      _gather_body,
      out_type=[
          jax.ShapeDtypeStruct((B, H), jnp.float32),
          jax.ShapeDtypeStruct((B, H), jnp.float32),
      ],
      mesh=plsc.VectorSubcoreMesh(core_axis_name="c", subcore_axis_name="s"),
      scratch_types=[
          pltpu.VMEM((NCH, CH), jnp.int32),
          pltpu.VMEM((NCH, CH), jnp.int32),
          pltpu.VMEM((ROWS_W, H), jnp.float32),
          pltpu.SemaphoreType.DMA,
      ],
  )


def _tc_body(t_ref, m_ref, xt_ref, w1_ref, b1_ref, w2_ref, b2_ref,
             pcw_ref, pcb_ref, ow_ref, ob_ref, o_ref):
  f32 = jnp.float32
  dgt = lambda a, b: lax.dot_general(
      a, b, (((0,), (0,)), ((), ())), preferred_element_type=f32,
      precision=lax.Precision.HIGHEST)
  wt = ow_ref[0:H, :]
  wm = ow_ref[H:2 * H, :]
  ws = ow_ref[2 * H:3 * H, :]
  wp = ow_ref[3 * H:4 * H, :]
  xb = xt_ref[:]
  h = dgt(xb, w1_ref[:]) + b1_ref[:]
  h = h * (1.0 / (1.0 + jnp.exp(-h)))
  s_emb = jnp.dot(h, w2_ref[:], preferred_element_type=f32) + b2_ref[:]
  pcwp = jnp.dot(pcw_ref[:], wp, preferred_element_type=f32)
  pcbwp = jnp.dot(pcb_ref[:], wp, preferred_element_type=f32)
  sel = lax.broadcasted_iota(jnp.int32, (8, 1), 0) == 4
  pmat = jnp.where(sel, jnp.broadcast_to(pcwp, (8, H)), 0.0)
  acc = jnp.dot(t_ref[:], wt, preferred_element_type=f32)
  acc += jnp.dot(m_ref[:], wm, preferred_element_type=f32)
  acc += jnp.dot(s_emb, ws, preferred_element_type=f32)
  acc += dgt(xb, pmat)
  o_ref[:] = acc + pcbwp + ob_ref[:]


def _tc_call(t_emb, sem_emb, xt, w1x, b1p, w2p, b2, pc_w, pc_b, out_w,
             out_b, interpret=False):
  nb = B // BT
  row = lambda i: (i, 0)
  col = lambda i: (0, i)
  rep = lambda i: (0, 0)
  return pl.pallas_call(
      _tc_body,
      grid=(nb,),
      in_specs=[
          pl.BlockSpec((BT, H), row),
          pl.BlockSpec((BT, H), row),
          pl.BlockSpec((8, BT), col),
          pl.BlockSpec((8, H), rep),
          pl.BlockSpec((1, H), rep),
          pl.BlockSpec((H, H), rep),
          pl.BlockSpec((1, H), rep),
          pl.BlockSpec((1, H), rep),
          pl.BlockSpec((1, H), rep),
          pl.BlockSpec((4 * H, H), rep),
          pl.BlockSpec((1, H), rep),
      ],
      out_specs=pl.BlockSpec((BT, H), row),
      out_shape=jax.ShapeDtypeStruct((B, H), jnp.float32),
      interpret=interpret,
  )(t_emb, sem_emb, xt, w1x, b1p, w2p, b2, pc_w, pc_b, out_w, out_b)


def kernel(type_ids, semantic_ids, shape_vecs, param_counts, type_table,
           sem_table, shape_W1, shape_b1, shape_W2, shape_b2, pc_W, pc_b,
           out_W, out_b):
  tids = type_ids.astype(jnp.int32).reshape(NW, NCH, CH)
  sids = semantic_ids.astype(jnp.int32).reshape(NW, NCH, CH)
  t_emb, sem_emb = _gather_call()(type_table, tids, sem_table, sids)
  xt = (jnp.zeros((8, B), jnp.float32)
        .at[0:4, :].set(shape_vecs.T)
        .at[4, :].set(param_counts))
  w1x = jnp.zeros((8, H), jnp.float32).at[0:4, :H // 2].set(shape_W1)
  b1p = jnp.zeros((1, H), jnp.float32).at[:, :H // 2].set(shape_b1)
  w2p = jnp.zeros((H, H), jnp.float32).at[:H // 2, :].set(shape_W2)
  return _tc_call(t_emb, sem_emb, xt, w1x, b1p,
                  w2p, shape_b2.reshape(1, H), pc_W, pc_b.reshape(1, H), out_W,
                  out_b.reshape(1, H))


# speedup vs baseline: 1.3726x; 1.3726x over previous
import functools

import jax
import jax.numpy as jnp
from jax import lax
from jax.experimental import pallas as pl
from jax.experimental.pallas import tpu as pltpu
from jax.experimental.pallas import tpu_sc as plsc

B = 16384
H = 128
NC = 2
NS = 16
NW = NC * NS
ROWS_W = B // NW
CH = 128
NCH = ROWS_W // CH

BT = 2048


def _gather_body(tt_hbm, tid_hbm, st_hbm, sid_hbm, t_out, s_out,
                 tidx_v, sidx_v, rows_v, dsem):
  wid = lax.axis_index("s") * NC + lax.axis_index("c")
  base = wid * ROWS_W
  pltpu.sync_copy(sid_hbm.at[wid], sidx_v)
  pltpu.sync_copy(tid_hbm.at[wid], tidx_v)
  waits = []
  for c in range(NCH):
    waits.append(pltpu.async_copy(
        st_hbm.at[sidx_v.at[c]], rows_v.at[pl.ds(c * CH, CH)], dsem))
  for w in waits:
    w.wait()
  pltpu.sync_copy(rows_v, s_out.at[pl.ds(base, ROWS_W)])
  waits = []
  for c in range(NCH):
    waits.append(pltpu.async_copy(
        tt_hbm.at[tidx_v.at[c]], rows_v.at[pl.ds(c * CH, CH)], dsem))
  for w in waits:
    w.wait()
  pltpu.sync_copy(rows_v, t_out.at[pl.ds(base, ROWS_W)])


@functools.cache
def _gather_call():
  return pl.kernel(
      _gather_body,
      out_type=[
          jax.ShapeDtypeStruct((B, H), jnp.float32),
          jax.ShapeDtypeStruct((B, H), jnp.float32),
      ],
      mesh=plsc.VectorSubcoreMesh(core_axis_name="c", subcore_axis_name="s"),
      scratch_types=[
          pltpu.VMEM((NCH, CH), jnp.int32),
          pltpu.VMEM((NCH, CH), jnp.int32),
          pltpu.VMEM((ROWS_W, H), jnp.float32),
          pltpu.SemaphoreType.DMA,
      ],
  )


def _tc_body(t_ref, m_ref, xt_ref, w1_ref, b1_ref, w2_ref, b2_ref,
             pcw_ref, pcb_ref, ow_ref, ob_ref, o_ref):
  f32 = jnp.float32
  wt = ow_ref[0:H, :]
  wm = ow_ref[H:2 * H, :]
  ws = ow_ref[2 * H:3 * H, :]
  wp = ow_ref[3 * H:4 * H, :]
  x = jnp.transpose(xt_ref[:])
  h = jnp.dot(x, w1_ref[:], preferred_element_type=f32) + b1_ref[:]
  h = h * (1.0 / (1.0 + jnp.exp(-h)))
  s_emb = jnp.dot(h, w2_ref[:], preferred_element_type=f32) + b2_ref[:]
  p_emb = x[:, 4:5] * pcw_ref[:] + pcb_ref[:]
  acc = jnp.dot(t_ref[:], wt, preferred_element_type=f32)
  acc += jnp.dot(m_ref[:], wm, preferred_element_type=f32)
  acc += jnp.dot(s_emb, ws, preferred_element_type=f32)
  acc += jnp.dot(p_emb, wp, preferred_element_type=f32)
  o_ref[:] = acc + ob_ref[:]


def _tc_call(t_emb, sem_emb, xt, w1x, b1p, w2p, b2, pc_w, pc_b, out_w,
             out_b, interpret=False):
  nb = B // BT
  row = lambda i: (i, 0)
  col = lambda i: (0, i)
  rep = lambda i: (0, 0)
  return pl.pallas_call(
      _tc_body,
      grid=(nb,),
      in_specs=[
          pl.BlockSpec((BT, H), row),
          pl.BlockSpec((BT, H), row),
          pl.BlockSpec((8, BT), col),
          pl.BlockSpec((8, H), rep),
          pl.BlockSpec((1, H), rep),
          pl.BlockSpec((H, H), rep),
          pl.BlockSpec((1, H), rep),
          pl.BlockSpec((1, H), rep),
          pl.BlockSpec((1, H), rep),
          pl.BlockSpec((4 * H, H), rep),
          pl.BlockSpec((1, H), rep),
      ],
      out_specs=pl.BlockSpec((BT, H), row),
      out_shape=jax.ShapeDtypeStruct((B, H), jnp.float32),
      interpret=interpret,
  )(t_emb, sem_emb, xt, w1x, b1p, w2p, b2, pc_w, pc_b, out_w, out_b)


def kernel(type_ids, semantic_ids, shape_vecs, param_counts, type_table,
           sem_table, shape_W1, shape_b1, shape_W2, shape_b2, pc_W, pc_b,
           out_W, out_b):
  tids = type_ids.astype(jnp.int32).reshape(NW, NCH, CH)
  sids = semantic_ids.astype(jnp.int32).reshape(NW, NCH, CH)
  t_emb, sem_emb = _gather_call()(type_table, tids, sem_table, sids)
  xt = (jnp.zeros((8, B), jnp.float32)
        .at[0:4, :].set(shape_vecs.T)
        .at[4, :].set(param_counts))
  w1x = jnp.zeros((8, H), jnp.float32).at[0:4, :H // 2].set(shape_W1)
  b1p = jnp.zeros((1, H), jnp.float32).at[:, :H // 2].set(shape_b1)
  w2p = jnp.zeros((H, H), jnp.float32).at[:H // 2, :].set(shape_W2)
  return _tc_call(t_emb, sem_emb, xt, w1x, b1p,
                  w2p, shape_b2.reshape(1, H), pc_W, pc_b.reshape(1, H), out_W,
                  out_b.reshape(1, H))
